# baseline (speedup 1.0000x reference)
import functools

import jax
import jax.numpy as jnp
from jax import lax
from jax.experimental import pallas as pl
from jax.experimental.pallas import tpu as pltpu
from jax.experimental.pallas import tpu_sc as plsc

_LANE = 16
_BT = 128
_HT = 8
_HHALF = 4
_UNIT = _HHALF * _BT


def _gather_sc(idx5, table, D, TR, TC):
    H = TR * _HT
    G = D // _HT
    n_units = TR * 2
    assert n_units >= 6 and n_units % 2 == 0

    mesh = plsc.VectorSubcoreMesh(core_axis_name="c", subcore_axis_name="s")
    info = plsc.get_sparse_core_info()
    num_cores = info.num_cores

    @functools.partial(
        pl.kernel,
        out_type=jax.ShapeDtypeStruct((H, G, TC, _HT, _BT), jnp.float32),
        mesh=mesh,
        scratch_types=[
            [pltpu.VMEM((_UNIT,), jnp.int32) for _ in range(2)],
            [pltpu.VMEM((_UNIT, D), jnp.float32) for _ in range(2)],
            [pltpu.VMEM((_HHALF, G, _HT, _BT), jnp.float32) for _ in range(2)],
            [pltpu.SemaphoreType.DMA for _ in range(2)],
            [pltpu.SemaphoreType.DMA for _ in range(2)],
            [pltpu.SemaphoreType.DMA for _ in range(2)],
        ],
        compiler_params=pltpu.CompilerParams(
            use_tc_tiling_on_sc=False, needs_layout_passes=False
        ),
    )
    def body(idx_hbm, table_hbm, z_hbm, idx_v, rows_v, zt_v, isem, gsem, wsem):
        w = lax.axis_index("s") * num_cores + lax.axis_index("c")

        def idx_load(u, b):
            R = u // 2
            half = u % 2
            pltpu.make_async_copy(
                idx_hbm.at[R, w, pl.ds(half * _UNIT, _UNIT)],
                idx_v[b], isem[b],
            ).start()

        def idx_wait(b):
            pltpu.make_async_copy(
                idx_hbm.at[0, w, pl.ds(0, _UNIT)], idx_v[b], isem[b]
            ).wait()

        def gather_start(b):
            pltpu.make_async_copy(
                table_hbm.at[idx_v[b]], rows_v[b], gsem[b]
            ).start()

        def gather_wait(b):
            pltpu.make_async_copy(
                table_hbm.at[idx_v[b]], rows_v[b], gsem[b]
            ).wait()

        def transpose(b):
            @pl.loop(0, _HHALF)
            def _(r_):
                @pl.loop(0, D)
                def _(d):
                    g = d // _HT
                    rr = d % _HT
                    colv = jnp.full((_LANE,), d, jnp.int32)
                    base = r_ * _BT
                    for c0 in range(0, _BT, _LANE):
                        rowv = base + c0 + lax.iota(jnp.int32, _LANE)
                        vec = plsc.load_gather(rows_v[b], [rowv, colv])
                        zt_v[b][r_, g, rr, pl.ds(c0, _LANE)] = vec

        def write_start(u, b):
            R = u // 2
            half = u % 2
            for r_ in range(_HHALF):
                pltpu.make_async_copy(
                    zt_v[b].at[r_],
                    z_hbm.at[R * _HT + half * _HHALF + r_, :, w],
                    wsem[b],
                ).start()

        def write_drain(b):
            for r_ in range(_HHALF):
                pltpu.make_async_copy(
                    zt_v[b].at[r_], z_hbm.at[0, :, w], wsem[b]
                ).wait()

        def step(u, b, drain, load_next):
            p = 1 - b
            idx_wait(b)
            gather_start(b)
            gather_wait(p)
            if load_next:
                idx_load(u + 1, p)
            if drain:
                write_drain(p)
            transpose(p)
            write_start(u - 1, p)

        pltpu.sync_copy(idx_hbm.at[0, w, pl.ds(0, _UNIT)], idx_v[0])
        gather_start(0)
        idx_load(1, 1)
        step(1, 1, drain=False, load_next=True)
        step(2, 0, drain=False, load_next=True)
        step(3, 1, drain=True, load_next=True)

        @pl.loop(4, n_units - 2, step=2)
        def _(u0):
            step(u0, 0, drain=True, load_next=True)
            step(u0 + 1, 1, drain=True, load_next=True)

        step(n_units - 2, 0, drain=True, load_next=True)
        step(n_units - 1, 1, drain=True, load_next=False)
        gather_wait(1)
        write_drain(1)
        transpose(1)
        write_start(n_units - 1, 1)
        write_drain(0)
        write_drain(1)

    return body(idx5, table)


def kernel(batch_data, table):
    B, H = batch_data.shape
    V, D = table.shape
    TR = H // _HT
    TC = B // _BT
    idx5 = (
        batch_data.T.reshape(TR, _HT, TC, _BT)
        .transpose(0, 2, 1, 3)
        .reshape(TR, TC, _HT * _BT)
    )
    z = _gather_sc(idx5, table, D, TR, TC)
    out = z.transpose(2, 4, 0, 1, 3).reshape(B, H, D)
    return out


# speedup vs baseline: 1.5822x; 1.5822x over previous
import functools

import jax
import jax.numpy as jnp
from jax import lax
from jax.experimental import pallas as pl
from jax.experimental.pallas import tpu as pltpu
from jax.experimental.pallas import tpu_sc as plsc

_LANE = 16
_BT = 128
_HT = 8
_HHALF = 4
_UNIT = _HHALF * _BT


def _gather_sc(idx5, table, D, TR, TC):
    H = TR * _HT
    G = D // _HT
    n_units = TR * 2
    assert n_units >= 6 and n_units % 2 == 0

    mesh = plsc.VectorSubcoreMesh(core_axis_name="c", subcore_axis_name="s")
    info = plsc.get_sparse_core_info()
    num_cores = info.num_cores

    @functools.partial(
        pl.kernel,
        out_type=jax.ShapeDtypeStruct((H, G, TC, _HT, _BT), jnp.float32),
        mesh=mesh,
        scratch_types=[
            [pltpu.VMEM((_UNIT,), jnp.int32) for _ in range(2)],
            [pltpu.VMEM((_UNIT, D), jnp.float32) for _ in range(2)],
            [pltpu.VMEM((_HHALF, G, _HT, _BT + 1), jnp.float32) for _ in range(2)],
            [pltpu.SemaphoreType.DMA for _ in range(2)],
            [pltpu.SemaphoreType.DMA for _ in range(2)],
            [pltpu.SemaphoreType.DMA for _ in range(2)],
        ],
        compiler_params=pltpu.CompilerParams(
            use_tc_tiling_on_sc=False, needs_layout_passes=False
        ),
    )
    def body(idx_hbm, table_hbm, z_hbm, idx_v, rows_v, zt_v, isem, gsem, wsem):
        w = lax.axis_index("s") * num_cores + lax.axis_index("c")
        dlane = lax.iota(jnp.int32, _LANE)
        gv = [(dlane + d0) // _HT for d0 in range(0, D, _LANE)]
        rv = [(dlane + d0) % _HT for d0 in range(0, D, _LANE)]

        def idx_load(u, b):
            R = u // 2
            half = u % 2
            pltpu.make_async_copy(
                idx_hbm.at[R, w, pl.ds(half * _UNIT, _UNIT)],
                idx_v[b], isem[b],
            ).start()

        def idx_wait(b):
            pltpu.make_async_copy(
                idx_hbm.at[0, w, pl.ds(0, _UNIT)], idx_v[b], isem[b]
            ).wait()

        def gather_start(b):
            pltpu.make_async_copy(
                table_hbm.at[idx_v[b]], rows_v[b], gsem[b]
            ).start()

        def gather_wait(b):
            pltpu.make_async_copy(
                table_hbm.at[idx_v[b]], rows_v[b], gsem[b]
            ).wait()

        def transpose(b):
            @pl.loop(0, _UNIT, unroll=4)
            def _(q):
                r_ = q // _BT
                c = q % _BT
                rf = jnp.full((_LANE,), r_, jnp.int32)
                cf = jnp.full((_LANE,), c, jnp.int32)
                for k in range(D // _LANE):
                    vec = rows_v[b][q, pl.ds(k * _LANE, _LANE)]
                    plsc.store_scatter(zt_v[b], [rf, gv[k], rv[k], cf], vec)

        def write_start(u, b):
            R = u // 2
            half = u % 2
            for r_ in range(_HHALF):
                pltpu.make_async_copy(
                    zt_v[b].at[r_, :, :, pl.ds(0, _BT)],
                    z_hbm.at[R * _HT + half * _HHALF + r_, :, w],
                    wsem[b],
                ).start()

        def write_drain(b):
            for r_ in range(_HHALF):
                pltpu.make_async_copy(
                    zt_v[b].at[r_, :, :, pl.ds(0, _BT)],
                    z_hbm.at[0, :, w], wsem[b],
                ).wait()

        def step(u, b, drain, load_next):
            p = 1 - b
            idx_wait(b)
            gather_start(b)
            gather_wait(p)
            if load_next:
                idx_load(u + 1, p)
            if drain:
                write_drain(p)
            transpose(p)
            write_start(u - 1, p)

        pltpu.sync_copy(idx_hbm.at[0, w, pl.ds(0, _UNIT)], idx_v[0])
        gather_start(0)
        idx_load(1, 1)
        step(1, 1, drain=False, load_next=True)
        step(2, 0, drain=False, load_next=True)
        step(3, 1, drain=True, load_next=True)

        @pl.loop(4, n_units - 2, step=2)
        def _(u0):
            step(u0, 0, drain=True, load_next=True)
            step(u0 + 1, 1, drain=True, load_next=True)

        step(n_units - 2, 0, drain=True, load_next=True)
        step(n_units - 1, 1, drain=True, load_next=False)
        gather_wait(1)
        write_drain(1)
        transpose(1)
        write_start(n_units - 1, 1)
        write_drain(0)
        write_drain(1)

    return body(idx5, table)


def kernel(batch_data, table):
    B, H = batch_data.shape
    V, D = table.shape
    TR = H // _HT
    TC = B // _BT
    idx5 = (
        batch_data.T.reshape(TR, _HT, TC, _BT)
        .transpose(0, 2, 1, 3)
        .reshape(TR, TC, _HT * _BT)
    )
    z = _gather_sc(idx5, table, D, TR, TC)
    out = z.transpose(2, 4, 0, 1, 3).reshape(B, H, D)
    return out
